# Initial kernel scaffold; baseline (speedup 1.0000x reference)
import jax
import jax.numpy as jnp
from jax.experimental import pallas as pl


def kernel(inputs, adj, W, b):
    raise NotImplementedError("write your pallas kernel here")


# speedup vs baseline: 9.5557x; 9.5557x over previous
import jax
import jax.numpy as jnp
from jax import lax
from jax.experimental import pallas as pl
from jax.experimental.pallas import tpu as pltpu
from jax.experimental.pallas import tpu_sc as plsc

N = 10000
E = 160000
D = 256
Q = 128
NQ = D // Q
NC = 2
NS = 16

_MESH = dict(core_axis_name="c", subcore_axis_name="s")

NT = 624
WBC = 208
NWB = NT // WBC
TAIL_BASE = NS * NT
TAIL = N - TAIL_BASE

DEG_W = 16


def _node_chunks(body):
    s = lax.axis_index("s")
    nb = s * NT
    for t in range(NWB):
        body(nb + t * WBC, WBC)

    @pl.when(s == NS - 1)
    def _():
        body(TAIL_BASE, TAIL)


E_TILE_DEG = E // (NC * NS)
DEG_CHUNK = 40
DEG_NCHUNK = E_TILE_DEG // DEG_CHUNK


def _deg_body(dst_hbm, degp_hbm, idxbuf, ones, zbuf, degacc):
    c = lax.axis_index("c")
    s = lax.axis_index("s")
    z16 = jnp.zeros((16,), jnp.float32)
    o16 = jnp.ones((16,), jnp.float32)

    def fill_ones(i, _):
        ones[i, :] = o16
        return 0

    lax.fori_loop(0, DEG_CHUNK, fill_ones, 0)

    def fill_zero(i, _):
        zbuf[i, :] = z16
        return 0

    lax.fori_loop(0, WBC, fill_zero, 0)

    def zero_deg(n0, m):
        pltpu.sync_copy(zbuf.at[pl.ds(0, m)], degacc.at[pl.ds(n0, m)])

    _node_chunks(zero_deg)
    plsc.subcore_barrier()

    base = (c * NS + s) * E_TILE_DEG

    def chunk(k, _):
        pltpu.sync_copy(dst_hbm.at[pl.ds(base + k * DEG_CHUNK, DEG_CHUNK)],
                        idxbuf)
        pltpu.sync_copy(ones, degacc.at[idxbuf], add=True)
        return 0

    lax.fori_loop(0, DEG_NCHUNK, chunk, 0)
    plsc.subcore_barrier()

    def dump_deg(n0, m):
        pltpu.sync_copy(degacc.at[pl.ds(n0, m)], zbuf.at[pl.ds(0, m)])
        pltpu.sync_copy(zbuf.at[pl.ds(0, m)], degp_hbm.at[c, pl.ds(n0, m)])

    _node_chunks(dump_deg)


_deg_call = pl.kernel(
    _deg_body,
    out_type=jax.ShapeDtypeStruct((NC, N, DEG_W), jnp.float32),
    mesh=plsc.VectorSubcoreMesh(**_MESH),
    scratch_types=[
        pltpu.VMEM((DEG_CHUNK,), jnp.int32),
        pltpu.VMEM((DEG_CHUNK, DEG_W), jnp.float32),
        pltpu.VMEM((WBC, DEG_W), jnp.float32),
        pltpu.VMEM_SHARED((N, DEG_W), jnp.float32),
    ],
)

BM = 2000
NB = N // BM


def _mm_body(x_ref, w_ref, degp_ref, y_ref, disq_ref):
    acc = jnp.dot(x_ref[...], w_ref[...][0],
                  preferred_element_type=jnp.float32)
    d = degp_ref[...]
    deg = d[0, :, 0:1] + d[1, :, 0:1] + 1.0
    dq = lax.rsqrt(deg)
    y_ref[...] = acc * dq
    disq_ref[...] = jnp.broadcast_to(dq, (BM, DEG_W))


_mm_call = pl.pallas_call(
    _mm_body,
    grid=(NB, NQ),
    in_specs=[
        pl.BlockSpec((BM, D), lambda i, h: (i, 0)),
        pl.BlockSpec((1, D, Q), lambda i, h: (h, 0, 0)),
        pl.BlockSpec((NC, BM, DEG_W), lambda i, h: (0, i, 0)),
    ],
    out_specs=[
        pl.BlockSpec((BM, Q), lambda i, h: (h * NB + i, 0)),
        pl.BlockSpec((BM, DEG_W), lambda i, h: (i, 0)),
    ],
    out_shape=[
        jax.ShapeDtypeStruct((NQ * N, Q), jnp.float32),
        jax.ShapeDtypeStruct((N, DEG_W), jnp.float32),
    ],
)

EC = 80
E_TILE = E // NS
NCH = E_TILE // EC


def _agg_body(y_hbm, src_hbm, dst_hbm, accq_hbm,
              srcbuf, dstbuf, rows, nbuf, acc):
    c = lax.axis_index("c")
    s = lax.axis_index("s")

    def seed_acc(q):
        def one(n0, m):
            pltpu.sync_copy(y_hbm.at[pl.ds(q * N + n0, m)],
                            nbuf.at[pl.ds(0, m)])
            pltpu.sync_copy(nbuf.at[pl.ds(0, m)], acc.at[pl.ds(n0, m)])
        _node_chunks(one)

    def dump_acc(q):
        def one(n0, m):
            pltpu.sync_copy(acc.at[pl.ds(n0, m)], nbuf.at[pl.ds(0, m)])
            pltpu.sync_copy(nbuf.at[pl.ds(0, m)],
                            accq_hbm.at[q, pl.ds(n0, m)])
        _node_chunks(one)

    ebase = s * E_TILE

    def edge_loop(q):
        off = jnp.full((16,), q * N, jnp.int32)

        def chunk(k, _):
            e0 = ebase + k * EC
            pltpu.sync_copy(src_hbm.at[pl.ds(e0, EC)], srcbuf)
            for t in range(EC // 16):
                sl = pl.ds(t * 16, 16)
                srcbuf[sl] = srcbuf[sl] + off
            pltpu.sync_copy(y_hbm.at[srcbuf], rows)
            pltpu.sync_copy(dst_hbm.at[pl.ds(e0, EC)], dstbuf)
            pltpu.sync_copy(rows, acc.at[dstbuf], add=True)
            return 0

        lax.fori_loop(0, NCH, chunk, 0)

    for p in range(NQ // NC):
        q = c * (NQ // NC) + p
        seed_acc(q)
        plsc.subcore_barrier()
        edge_loop(q)
        plsc.subcore_barrier()
        dump_acc(q)


_agg_call = pl.kernel(
    _agg_body,
    out_type=jax.ShapeDtypeStruct((NQ, N, Q), jnp.float32),
    mesh=plsc.VectorSubcoreMesh(**_MESH),
    scratch_types=[
        pltpu.VMEM((EC,), jnp.int32),
        pltpu.VMEM((EC,), jnp.int32),
        pltpu.VMEM((EC, Q), jnp.float32),
        pltpu.VMEM((WBC, Q), jnp.float32),
        pltpu.VMEM_SHARED((N, Q), jnp.float32),
    ],
)


def _epi_body(*refs):
    a_refs, (disq_ref, b_ref, out_ref) = refs[:NQ], refs[NQ:]
    a = jnp.concatenate([r[...][0] for r in a_refs], axis=-1)
    dq = disq_ref[...][:, 0:1]
    out_ref[...] = jnp.maximum(a * dq + b_ref[...], 0.0)


_epi_call = pl.pallas_call(
    _epi_body,
    grid=(NB,),
    in_specs=(
        [pl.BlockSpec((1, BM, Q), lambda i, q=q: (q, i, 0))
         for q in range(NQ)]
        + [
            pl.BlockSpec((BM, DEG_W), lambda i: (i, 0)),
            pl.BlockSpec((1, D), lambda i: (0, 0)),
        ]
    ),
    out_specs=pl.BlockSpec((BM, D), lambda i: (i, 0)),
    out_shape=jax.ShapeDtypeStruct((N, D), jnp.float32),
)


def kernel(inputs, adj, W, b):
    adj32 = adj.astype(jnp.int32)
    src = adj32[0]
    dst = adj32[1]
    degp = _deg_call(dst)
    w4 = W.reshape(D, NQ, Q).transpose(1, 0, 2)
    y4, disq = _mm_call(inputs, w4, degp)
    accq = _agg_call(y4, src, dst)
    b2 = b.astype(jnp.float32).reshape(1, D)
    return _epi_call(*([accq] * NQ), disq, b2)


# speedup vs baseline: 14.3233x; 1.4989x over previous
import jax
import jax.numpy as jnp
from jax import lax
from jax.experimental import pallas as pl
from jax.experimental.pallas import tpu as pltpu
from jax.experimental.pallas import tpu_sc as plsc

N = 10000
E = 160000
D = 256
Q = 128
NQ = D // Q
NC = 2
NS = 16

_MESH = dict(core_axis_name="c", subcore_axis_name="s")

NT = 624
WBC = 208
NWB = NT // WBC
TAIL_BASE = NS * NT
TAIL = N - TAIL_BASE

DEG_W = 16

ECB = 128
N_CHUNKS = E // ECB
EC = 80
N_ECH = E // EC
AGG_CH = N_ECH // NS


def _node_chunks(body):
    s = lax.axis_index("s")
    nb = s * NT
    for t in range(NWB):
        body(nb + t * WBC, WBC)

    @pl.when(s == NS - 1)
    def _():
        body(TAIL_BASE, TAIL)


DEG_CH = N_CHUNKS // (NC * NS)
DEG_BASE = NC * NS * DEG_CH


def _deg_body(dst_hbm, degp_hbm, idxbuf, ones, zbuf, ds0, ds1, degacc):
    c = lax.axis_index("c")
    s = lax.axis_index("s")
    wid = c * NS + s
    z16 = jnp.zeros((16,), jnp.float32)
    o16 = jnp.ones((16,), jnp.float32)
    sems = (ds0, ds1)

    def fill_ones(i, _):
        ones[i, :] = o16
        return 0

    lax.fori_loop(0, ECB, fill_ones, 0)

    def fill_zero(i, _):
        zbuf[i, :] = z16
        return 0

    lax.fori_loop(0, WBC, fill_zero, 0)

    def zero_deg(n0, m):
        pltpu.sync_copy(zbuf.at[pl.ds(0, m)], degacc.at[pl.ds(n0, m)])

    _node_chunks(zero_deg)
    plsc.subcore_barrier()

    def load_idx(k, b):
        e0 = (wid + NC * NS * k) * ECB
        pltpu.sync_copy(dst_hbm.at[pl.ds(e0, ECB)], idxbuf.at[b])

    def start_sc(b):
        pltpu.async_copy(ones, degacc.at[idxbuf.at[b]], sems[b], add=True)

    def wait_sc(b):
        pltpu.make_async_copy(ones, degacc.at[idxbuf.at[b]], sems[b]).wait()

    load_idx(0, 0)
    start_sc(0)
    load_idx(1, 1)

    def pair(k2, _):
        k = 2 * k2 - 1
        start_sc(1)
        wait_sc(0)
        load_idx(k + 1, 0)
        start_sc(0)
        wait_sc(1)
        load_idx(k + 2, 1)
        return 0

    lax.fori_loop(1, 19, pair, 0)
    start_sc(1)
    wait_sc(0)
    start_sc(0)
    wait_sc(1)
    wait_sc(0)

    @pl.when(wid < NC)
    def _():
        e0 = (DEG_BASE + wid) * ECB
        pltpu.sync_copy(dst_hbm.at[pl.ds(e0, ECB)], idxbuf.at[0])
        pltpu.sync_copy(ones, degacc.at[idxbuf.at[0]], add=True)

    plsc.subcore_barrier()

    def dump_deg(n0, m):
        pltpu.sync_copy(degacc.at[pl.ds(n0, m)], zbuf.at[pl.ds(0, m)])
        pltpu.sync_copy(zbuf.at[pl.ds(0, m)], degp_hbm.at[c, pl.ds(n0, m)])

    _node_chunks(dump_deg)


_deg_call = pl.kernel(
    _deg_body,
    out_type=jax.ShapeDtypeStruct((NC, N, DEG_W), jnp.float32),
    mesh=plsc.VectorSubcoreMesh(**_MESH),
    scratch_types=[
        pltpu.VMEM((2, ECB), jnp.int32),
        pltpu.VMEM((ECB, DEG_W), jnp.float32),
        pltpu.VMEM((WBC, DEG_W), jnp.float32),
        pltpu.SemaphoreType.DMA,
        pltpu.SemaphoreType.DMA,
        pltpu.VMEM_SHARED((N, DEG_W), jnp.float32),
    ],
)

BM = 2000
NB = N // BM


def _mm_body(x_ref, w_ref, degp_ref, y_ref, disq_ref):
    acc = jnp.dot(x_ref[...], w_ref[...][0],
                  preferred_element_type=jnp.float32)
    d = degp_ref[...]
    deg = d[0, :, 0:1] + d[1, :, 0:1] + 1.0
    dq = lax.rsqrt(deg)
    y_ref[...] = acc * dq
    disq_ref[...] = jnp.broadcast_to(dq, (BM, DEG_W))


_mm_call = pl.pallas_call(
    _mm_body,
    grid=(NB, NQ),
    in_specs=[
        pl.BlockSpec((BM, D), lambda i, h: (i, 0)),
        pl.BlockSpec((1, D, Q), lambda i, h: (h, 0, 0)),
        pl.BlockSpec((NC, BM, DEG_W), lambda i, h: (0, i, 0)),
    ],
    out_specs=[
        pl.BlockSpec((BM, Q), lambda i, h: (h * NB + i, 0)),
        pl.BlockSpec((BM, DEG_W), lambda i, h: (i, 0)),
    ],
    out_shape=[
        jax.ShapeDtypeStruct((NQ * N, Q), jnp.float32),
        jax.ShapeDtypeStruct((N, DEG_W), jnp.float32),
    ],
)


def _agg_body(y_hbm, src_hbm, dst_hbm, accq_hbm,
              srcbuf, dstbuf, rows, nbuf, gs0, gs1, ss0, ss1, acc):
    c = lax.axis_index("c")
    s = lax.axis_index("s")
    gsems = (gs0, gs1)
    ssems = (ss0, ss1)

    def seed_acc(q):
        def one(n0, m):
            pltpu.sync_copy(y_hbm.at[pl.ds(q * N + n0, m)],
                            nbuf.at[pl.ds(0, m)])
            pltpu.sync_copy(nbuf.at[pl.ds(0, m)], acc.at[pl.ds(n0, m)])
        _node_chunks(one)

    def dump_acc(q):
        def one(n0, m):
            pltpu.sync_copy(acc.at[pl.ds(n0, m)], nbuf.at[pl.ds(0, m)])
            pltpu.sync_copy(nbuf.at[pl.ds(0, m)],
                            accq_hbm.at[q, pl.ds(n0, m)])
        _node_chunks(one)

    off = jnp.full((16,), c * N, jnp.int32)

    def issue(k, b):
        e0 = (s + NS * k) * EC
        pltpu.sync_copy(src_hbm.at[pl.ds(e0, EC)], srcbuf.at[b])
        for t in range(EC // 16):
            sl = pl.ds(t * 16, 16)
            srcbuf[b, sl] = srcbuf[b, sl] + off
        pltpu.async_copy(y_hbm.at[srcbuf.at[b]], rows.at[b], gsems[b])
        pltpu.sync_copy(dst_hbm.at[pl.ds(e0, EC)], dstbuf.at[b])

    def wait_gather(b):
        pltpu.make_async_copy(y_hbm.at[srcbuf.at[b]], rows.at[b],
                              gsems[b]).wait()

    def start_scatter(b):
        pltpu.async_copy(rows.at[b], acc.at[dstbuf.at[b]], ssems[b],
                         add=True)

    def wait_scatter(b):
        pltpu.make_async_copy(rows.at[b], acc.at[dstbuf.at[b]],
                              ssems[b]).wait()

    seed_acc(c)
    plsc.subcore_barrier()

    issue(0, 0)
    wait_gather(0)
    start_scatter(0)
    issue(1, 1)

    def pair(k2, _):
        k = 2 * k2 - 1
        wait_gather(1)
        start_scatter(1)
        wait_scatter(0)
        issue(k + 1, 0)
        wait_gather(0)
        start_scatter(0)
        wait_scatter(1)
        issue(k + 2, 1)
        return 0

    lax.fori_loop(1, 62, pair, 0)
    wait_gather(1)
    start_scatter(1)
    wait_scatter(0)
    issue(124, 0)
    wait_gather(0)
    start_scatter(0)
    wait_scatter(1)
    wait_scatter(0)

    plsc.subcore_barrier()
    dump_acc(c)


_agg_call = pl.kernel(
    _agg_body,
    out_type=jax.ShapeDtypeStruct((NQ, N, Q), jnp.float32),
    mesh=plsc.VectorSubcoreMesh(**_MESH),
    scratch_types=[
        pltpu.VMEM((2, EC), jnp.int32),
        pltpu.VMEM((2, EC), jnp.int32),
        pltpu.VMEM((2, EC, Q), jnp.float32),
        pltpu.VMEM((WBC, Q), jnp.float32),
        pltpu.SemaphoreType.DMA,
        pltpu.SemaphoreType.DMA,
        pltpu.SemaphoreType.DMA,
        pltpu.SemaphoreType.DMA,
        pltpu.VMEM_SHARED((N, Q), jnp.float32),
    ],
)


def _epi_body(*refs):
    a_refs, (disq_ref, b_ref, out_ref) = refs[:NQ], refs[NQ:]
    a = jnp.concatenate([r[...][0] for r in a_refs], axis=-1)
    dq = disq_ref[...][:, 0:1]
    out_ref[...] = jnp.maximum(a * dq + b_ref[...], 0.0)


_epi_call = pl.pallas_call(
    _epi_body,
    grid=(NB,),
    in_specs=(
        [pl.BlockSpec((1, BM, Q), lambda i, q=q: (q, i, 0))
         for q in range(NQ)]
        + [
            pl.BlockSpec((BM, DEG_W), lambda i: (i, 0)),
            pl.BlockSpec((1, D), lambda i: (0, 0)),
        ]
    ),
    out_specs=pl.BlockSpec((BM, D), lambda i: (i, 0)),
    out_shape=jax.ShapeDtypeStruct((N, D), jnp.float32),
)


def kernel(inputs, adj, W, b):
    adj32 = adj.astype(jnp.int32)
    src = adj32[0]
    dst = adj32[1]
    degp = _deg_call(dst)
    w4 = W.reshape(D, NQ, Q).transpose(1, 0, 2)
    y4, disq = _mm_call(inputs, w4, degp)
    accq = _agg_call(y4, src, dst)
    b2 = b.astype(jnp.float32).reshape(1, D)
    return _epi_call(accq, accq, disq, b2)
